# baseline (speedup 1.0000x reference)
import functools

import jax
import jax.numpy as jnp
from jax import lax
from jax.experimental import pallas as pl
from jax.experimental.pallas import tpu as pltpu
from jax.experimental.pallas import tpu_sc as plsc

_B = 16
_S = 4096
_M = 32768
_K = 65536

_NC = 2
_NS = 16
_NW = _NC * _NS
_SLOTS = _B * _S
_SPW = _SLOTS // _NW
_LANES = 16
_NSTREAM = 4
_RSTRIDE = _M // _NSTREAM + _LANES


def _sc_body(doc_hbm, end_hbm, md_hbm, emb_hbm, out_hbm,
             ds_v, es_v, fla_v, win_v, cbuf_v, cnt_v, idx_v, md_v, emb_v,
             out_v, md_sh, emb_sh, fl_sh, sem, sem_stage):
    sid = lax.axis_index("s")
    wid = sid * _NC + lax.axis_index("c")
    base = wid * _SPW
    mpt = _M // _NS

    @pl.when(sid == 0)
    def _stage_start():
        pltpu.async_copy(md_hbm, md_sh, sem_stage)
        pltpu.async_copy(emb_hbm, emb_sh, sem_stage)

    doc_dma = pltpu.async_copy(doc_hbm.at[pl.ds(sid * mpt, mpt)], ds_v, sem)
    end_dma = pltpu.async_copy(end_hbm.at[pl.ds(sid * mpt, mpt)], es_v, sem)

    iota = lax.iota(jnp.int32, _LANES)
    neg1 = jnp.full((_LANES,), -1, jnp.int32)
    wid_splat = jnp.full((_LANES,), 0, jnp.int32) + wid

    @plsc.parallel_loop(0, _SPW // _LANES, unroll=8)
    def init_step(j):
        win_v[pl.ds(j * _LANES, _LANES)] = neg1

    doc_dma.wait()
    end_dma.wait()

    with jax.named_scope("flatten"):
        @plsc.parallel_loop(0, mpt // _LANES, unroll=8)
        def flat_step(j):
            d = ds_v[pl.ds(j * _LANES, _LANES)]
            e = es_v[pl.ds(j * _LANES, _LANES)]
            ds_v[pl.ds(j * _LANES, _LANES)] = (d << 12) | e

        pltpu.sync_copy(ds_v, fl_sh.at[pl.ds(sid * mpt, mpt)])
    plsc.subcore_barrier()
    pltpu.sync_copy(fl_sh, fla_v)

    with jax.named_scope("scan_compact"):
        @plsc.parallel_loop(0, _M // _LANES, unroll=16,
                            carry=(jnp.int32(0), iota))
        def scan_step(g, carry):
            cnt, mvec = carry
            flat = fla_v[pl.ds(g * _LANES, _LANES)]
            gk = (flat << 15) | mvec
            msk = (gk >> 26) == wid_splat
            plsc.store_compressed(cbuf_v.at[pl.ds(cnt, _LANES)], gk, mask=msk)
            cnt = cnt + plsc.all_reduce_population_count(msk)[0]
            return cnt, mvec + 16

        cnt, _ = scan_step

        plsc.store_scatter(cbuf_v, [cnt + iota], neg1)
    cnts = [cnt]

    rot_key = (iota + (_LANES - 1)) & (_LANES - 1)
    last_lane = iota == (_LANES - 1)

    with jax.named_scope("resolve"):
        for k in range(len(cnts)):
            def resolve_step(g, _, _k=k):
                gk = cbuf_v[pl.ds(_k * _RSTRIDE + g * _LANES, _LANES)]
                (gk,) = lax.sort((gk,), dimension=0, num_keys=1)
                q = gk >> 15
                nxt = plsc.sort_key_val(rot_key, q)[1]
                valid = (gk >> 26) == wid_splat
                winner = valid & (last_lane | (q != nxt))
                loc = jnp.where(
                    winner, q - (jnp.full((_LANES,), 0, jnp.int32) + base), 0)
                plsc.store_scatter(win_v, [loc], gk & (_M - 1), mask=winner)
                return _

            lax.fori_loop(0, (cnts[k] + _LANES - 1) // _LANES, resolve_step, 0)

    with jax.named_scope("idx_prep"):
        @plsc.parallel_loop(0, _SPW // _LANES, unroll=8)
        def idx_step(j):
            w = win_v[pl.ds(j * _LANES, _LANES)]
            fallback = (base + j * _LANES + iota) & (_M - 1)
            idx_v[pl.ds(j * _LANES, _LANES)] = jnp.where(w >= 0, w, fallback)

    with jax.named_scope("gathers"):
        @pl.when(sid == 0)
        def _stage_wait():
            pltpu.make_async_copy(md_hbm, md_sh, sem_stage).wait()
            pltpu.make_async_copy(emb_hbm, emb_sh, sem_stage).wait()

        plsc.subcore_barrier()
        pltpu.async_copy(md_sh.at[idx_v], md_v, sem).wait()
        pltpu.async_copy(emb_sh.at[md_v], emb_v, sem).wait()

    zero = jnp.zeros((_LANES,), jnp.float32)

    with jax.named_scope("emit"):
        @plsc.parallel_loop(0, _SPW // _LANES, unroll=8)
        def out_step(j):
            w = win_v[pl.ds(j * _LANES, _LANES)]
            v = emb_v[pl.ds(j * _LANES, _LANES)]
            out_v[pl.ds(j * _LANES, _LANES)] = jnp.where(w >= 0, v, zero)

        pltpu.sync_copy(out_v, out_hbm.at[pl.ds(base, _SPW)])


_sc_call = functools.partial(
    pl.kernel,
    out_type=jax.ShapeDtypeStruct((_SLOTS,), jnp.float32),
    mesh=plsc.VectorSubcoreMesh(core_axis_name="c", subcore_axis_name="s"),
    compiler_params=pltpu.CompilerParams(needs_layout_passes=False),
    scratch_types=[
        pltpu.VMEM((_M // _NS,), jnp.int32),
        pltpu.VMEM((_M // _NS,), jnp.int32),
        pltpu.VMEM((_M,), jnp.int32),
        pltpu.VMEM((_SPW,), jnp.int32),
        pltpu.VMEM((_NSTREAM * _RSTRIDE,), jnp.int32),
        pltpu.VMEM((_LANES,), jnp.int32),
        pltpu.VMEM((_SPW,), jnp.int32),
        pltpu.VMEM((_SPW,), jnp.int32),
        pltpu.VMEM((_SPW,), jnp.float32),
        pltpu.VMEM((_SPW,), jnp.float32),
        pltpu.VMEM_SHARED((_M,), jnp.int32),
        pltpu.VMEM_SHARED((_K,), jnp.float32),
        pltpu.VMEM_SHARED((_M,), jnp.int32),
        pltpu.SemaphoreType.DMA,
        pltpu.SemaphoreType.DMA,
    ],
)(_sc_body)


def kernel(doc_idx, end_idx, match_data, embed):
    doc = doc_idx.astype(jnp.int32)
    end = end_idx.astype(jnp.int32)
    md = match_data.astype(jnp.int32)
    flat = _sc_call(doc, end, md, embed)
    return flat.reshape(_B, _S)


# speedup vs baseline: 1.0122x; 1.0122x over previous
import functools

import jax
import jax.numpy as jnp
from jax import lax
from jax.experimental import pallas as pl
from jax.experimental.pallas import tpu as pltpu
from jax.experimental.pallas import tpu_sc as plsc

_B = 16
_S = 4096
_M = 32768
_K = 65536

_NC = 2
_NS = 16
_NW = _NC * _NS
_SLOTS = _B * _S
_SPW = _SLOTS // _NW
_LANES = 16


def _sc_body(doc_hbm, end_hbm, md_hbm, emb_hbm, out_hbm,
             ds_v, es_v, fla_v, win_v, cbuf_v, idx_v, md_v, emb_v,
             out_v, md_sh, emb_sh, fl_sh, sem, sem_stage):
    sid = lax.axis_index("s")
    wid = sid * _NC + lax.axis_index("c")
    base = wid * _SPW
    mpt = _M // _NS

    @pl.when(sid == 0)
    def _stage_start():
        pltpu.async_copy(md_hbm, md_sh, sem_stage)
        pltpu.async_copy(emb_hbm, emb_sh, sem_stage)

    doc_dma = pltpu.async_copy(doc_hbm.at[pl.ds(sid * mpt, mpt)], ds_v, sem)
    end_dma = pltpu.async_copy(end_hbm.at[pl.ds(sid * mpt, mpt)], es_v, sem)

    iota = lax.iota(jnp.int32, _LANES)
    neg1 = jnp.full((_LANES,), -1, jnp.int32)
    wid_splat = jnp.full((_LANES,), 0, jnp.int32) + wid

    @plsc.parallel_loop(0, _SPW // _LANES, unroll=8)
    def init_step(j):
        win_v[pl.ds(j * _LANES, _LANES)] = neg1

    doc_dma.wait()
    end_dma.wait()

    with jax.named_scope("flatten"):
        @plsc.parallel_loop(0, mpt // _LANES, unroll=8)
        def flat_step(j):
            d = ds_v[pl.ds(j * _LANES, _LANES)]
            e = es_v[pl.ds(j * _LANES, _LANES)]
            ds_v[pl.ds(j * _LANES, _LANES)] = (d << 12) | e

        pltpu.sync_copy(ds_v, fl_sh.at[pl.ds(sid * mpt, mpt)])
    plsc.subcore_barrier()
    pltpu.sync_copy(fl_sh, fla_v)

    with jax.named_scope("scan_compact"):
        @plsc.parallel_loop(0, _M // _LANES, unroll=8,
                            carry=(jnp.int32(0), iota))
        def scan_step(g, carry):
            cnt, mvec = carry
            flat = fla_v[pl.ds(g * _LANES, _LANES)]
            gk = (flat << 15) | mvec
            msk = (gk >> 26) == wid_splat
            plsc.store_compressed(cbuf_v.at[pl.ds(cnt, _LANES)], gk, mask=msk)
            cnt = cnt + plsc.all_reduce_population_count(msk)[0]
            return cnt, mvec + 16

        cnt, _ = scan_step

        plsc.store_scatter(cbuf_v, [cnt + iota], neg1)

    rot_key = (iota + (_LANES - 1)) & (_LANES - 1)
    last_lane = iota == (_LANES - 1)

    with jax.named_scope("resolve"):
        def resolve_step(g, _):
            gk = cbuf_v[pl.ds(g * _LANES, _LANES)]
            (gk,) = lax.sort((gk,), dimension=0, num_keys=1)
            q = gk >> 15
            nxt = plsc.sort_key_val(rot_key, q)[1]
            valid = (gk >> 26) == wid_splat
            winner = valid & (last_lane | (q != nxt))
            loc = jnp.where(
                winner, q - (jnp.full((_LANES,), 0, jnp.int32) + base), 0)
            plsc.store_scatter(win_v, [loc], gk & (_M - 1), mask=winner)
            return _

        lax.fori_loop(0, (cnt + _LANES - 1) // _LANES, resolve_step, 0)

    with jax.named_scope("idx_prep"):
        @plsc.parallel_loop(0, _SPW // _LANES, unroll=8)
        def idx_step(j):
            w = win_v[pl.ds(j * _LANES, _LANES)]
            fallback = (base + j * _LANES + iota) & (_M - 1)
            idx_v[pl.ds(j * _LANES, _LANES)] = jnp.where(w >= 0, w, fallback)

    with jax.named_scope("gathers"):
        @pl.when(sid == 0)
        def _stage_wait():
            pltpu.make_async_copy(md_hbm, md_sh, sem_stage).wait()
            pltpu.make_async_copy(emb_hbm, emb_sh, sem_stage).wait()

        plsc.subcore_barrier()
        pltpu.async_copy(md_sh.at[idx_v], md_v, sem).wait()
        pltpu.async_copy(emb_sh.at[md_v], emb_v, sem).wait()

    zero = jnp.zeros((_LANES,), jnp.float32)

    with jax.named_scope("emit"):
        @plsc.parallel_loop(0, _SPW // _LANES, unroll=8)
        def out_step(j):
            w = win_v[pl.ds(j * _LANES, _LANES)]
            v = emb_v[pl.ds(j * _LANES, _LANES)]
            out_v[pl.ds(j * _LANES, _LANES)] = jnp.where(w >= 0, v, zero)

        pltpu.sync_copy(out_v, out_hbm.at[pl.ds(base, _SPW)])


_sc_call = functools.partial(
    pl.kernel,
    out_type=jax.ShapeDtypeStruct((_SLOTS,), jnp.float32),
    mesh=plsc.VectorSubcoreMesh(core_axis_name="c", subcore_axis_name="s"),
    compiler_params=pltpu.CompilerParams(needs_layout_passes=False),
    scratch_types=[
        pltpu.VMEM((_M // _NS,), jnp.int32),
        pltpu.VMEM((_M // _NS,), jnp.int32),
        pltpu.VMEM((_M,), jnp.int32),
        pltpu.VMEM((_SPW,), jnp.int32),
        pltpu.VMEM((_M + _LANES,), jnp.int32),
        pltpu.VMEM((_SPW,), jnp.int32),
        pltpu.VMEM((_SPW,), jnp.int32),
        pltpu.VMEM((_SPW,), jnp.float32),
        pltpu.VMEM((_SPW,), jnp.float32),
        pltpu.VMEM_SHARED((_M,), jnp.int32),
        pltpu.VMEM_SHARED((_K,), jnp.float32),
        pltpu.VMEM_SHARED((_M,), jnp.int32),
        pltpu.SemaphoreType.DMA,
        pltpu.SemaphoreType.DMA,
    ],
)(_sc_body)


def kernel(doc_idx, end_idx, match_data, embed):
    doc = doc_idx.astype(jnp.int32)
    end = end_idx.astype(jnp.int32)
    md = match_data.astype(jnp.int32)
    flat = _sc_call(doc, end, md, embed)
    return flat.reshape(_B, _S)
